# baseline (speedup 1.0000x reference)
import functools

import jax
import jax.numpy as jnp
from jax.experimental import pallas as pl
from jax.experimental.pallas import tpu as pltpu

_VARIANCES = (0.1, 0.2)
_THRESHOLD = 0.5
_NEGPOS_RATIO = 3

_NP = 8732
_NC = 72
_C = 128
_PAD = _NC * _C - _NP
_NB = 9
_CB = _NC // _NB


def _match_body(truths_ref, labels_ref, priors_ref, pos_ref, btid_ref,
                btov_s, btid_s, bval_s, bidx_s, num, num_objs):
    sl = pl.program_id(0)
    lane = jax.lax.broadcasted_iota(jnp.int32, (1, _C), 1)

    @pl.when(sl == 0)
    def _init():
        bval_s[...] = jnp.full((num_objs, num, _C), -1.0, jnp.float32)
        bidx_s[...] = jnp.zeros((num_objs, num, _C), jnp.int32)

    bvals = [bval_s[j] for j in range(num_objs)]
    bidxs = [bidx_s[j] for j in range(num_objs)]

    txs = []
    for j in range(num_objs):
        txs.append((truths_ref[:, 0, j][:, None], truths_ref[:, 1, j][:, None],
                    truths_ref[:, 2, j][:, None], truths_ref[:, 3, j][:, None]))

    for ch in range(_CB):
        gch = sl * _CB + ch
        pcx = priors_ref[0, gch, :][None, :]
        pcy = priors_ref[1, gch, :][None, :]
        pw = priors_ref[2, gch, :][None, :]
        ph = priors_ref[3, gch, :][None, :]
        px1 = pcx - pw / 2.0
        py1 = pcy - ph / 2.0
        px2 = pcx + pw / 2.0
        py2 = pcy + ph / 2.0
        parea = (px2 - px1) * (py2 - py1)
        idxrow = gch * _C + lane

        btov_c = jnp.full((num, _C), -1.0, jnp.float32)
        btid_c = jnp.zeros((num, _C), jnp.int32)
        for j in range(num_objs):
            tx1, ty1, tx2, ty2 = txs[j]
            ix = jnp.clip(jnp.minimum(px2, tx2) - jnp.maximum(px1, tx1),
                          0.0, None)
            iy = jnp.clip(jnp.minimum(py2, ty2) - jnp.maximum(py1, ty1),
                          0.0, None)
            inter = ix * iy
            tarea = (tx2 - tx1) * (ty2 - ty1)
            ov = inter / (tarea + parea - inter)
            upd = ov > btov_c
            btov_c = jnp.where(upd, ov, btov_c)
            btid_c = jnp.where(upd, j, btid_c)
            upd2 = ov > bvals[j]
            bvals[j] = jnp.where(upd2, ov, bvals[j])
            bidxs[j] = jnp.where(upd2, idxrow, bidxs[j])
        btov_s[gch] = btov_c
        btid_s[gch] = btid_c

    for j in range(num_objs):
        bval_s[j] = bvals[j]
        bidx_s[j] = bidxs[j]

    @pl.when(sl == _NB - 1)
    def _finalize():
        bps = []
        for j in range(num_objs):
            bv = bval_s[j]
            bi = bidx_s[j]
            m = jnp.max(bv, axis=1, keepdims=True)
            bps.append(jnp.min(jnp.where(bv == m, bi, jnp.int32(2**30)),
                               axis=1, keepdims=True))
        for gch in range(_NC):
            idxrow = gch * _C + lane
            b_ov = btov_s[gch]
            b_id = btid_s[gch]
            for j in range(num_objs):
                eq = idxrow == bps[j]
                b_ov = jnp.where(eq, 2.0, b_ov)
                b_id = jnp.where(eq, j, b_id)
            conf_t = jnp.zeros((num, _C), jnp.float32)
            for j in range(num_objs):
                conf_t = jnp.where(b_id == j, labels_ref[:, j][:, None],
                                   conf_t)
            conf_t = jnp.where(b_ov < _THRESHOLD, 0.0, conf_t)
            pos = (conf_t > 0.0) & (idxrow < _NP)
            pos_ref[gch] = pos.astype(jnp.float32)
            btid_ref[gch] = b_id


def _loss_body(truths_ref, priors_ref, loc_ref, conf_ref, pos_ref, btid_ref,
               out_ref, vA, acc, num, num_objs, num_classes):
    i = pl.program_id(0)
    lane = jax.lax.broadcasted_iota(jnp.int32, (1, _C), 1)

    @pl.when(i < _NB)
    def _phase1():
        sl = i

        @pl.when(sl == 0)
        def _init():
            acc[...] = jnp.zeros((3, num, _C), jnp.float32)

        npos_a = acc[0]
        ll_a = acc[1]
        cep_a = acc[2]
        for ch in range(_CB):
            gch = sl * _CB + ch
            s = jnp.exp(conf_ref[ch, 0])
            for c in range(1, num_classes):
                s = s + jnp.exp(conf_ref[ch, c])
            lse = jnp.log(s)

            pos = pos_ref[gch] > 0.0
            bt_id = btid_ref[gch]
            idxrow = gch * _C + lane

            npos_a = npos_a + jnp.where(pos, 1.0, 0.0)
            cep_a = cep_a + jnp.where(pos, lse - conf_ref[ch, 1], 0.0)
            vA[gch] = jnp.maximum(
                jnp.where(pos | (idxrow >= _NP), 0.0, lse - conf_ref[ch, 0]),
                0.0)

            pcx = priors_ref[0, gch, :][None, :]
            pcy = priors_ref[1, gch, :][None, :]
            pw = priors_ref[2, gch, :][None, :]
            ph = priors_ref[3, gch, :][None, :]
            mt = []
            for c4 in range(4):
                m_acc = jnp.zeros((num, _C), jnp.float32)
                for j in range(num_objs):
                    m_acc = jnp.where(bt_id == j,
                                      truths_ref[:, c4, j][:, None], m_acc)
                mt.append(m_acc)
            mx1, my1, mx2, my2 = mt
            g = (((mx1 + mx2) / 2.0 - pcx) / (_VARIANCES[0] * pw),
                 ((my1 + my2) / 2.0 - pcy) / (_VARIANCES[0] * ph),
                 jnp.log((mx2 - mx1) / pw) / _VARIANCES[1],
                 jnp.log((my2 - my1) / ph) / _VARIANCES[1])
            for c4 in range(4):
                d = loc_ref[ch, c4] - g[c4]
                ad = jnp.abs(d)
                sl1 = jnp.where(ad < 1.0, 0.5 * d * d, ad - 0.5)
                ll_a = ll_a + jnp.where(pos, sl1, 0.0)
        acc[0] = npos_a
        acc[1] = ll_a
        acc[2] = cep_a

    @pl.when(i == _NB)
    def _phase2():
        npos = jnp.sum(acc[0], axis=1, keepdims=True)
        ce_pos = jnp.sum(acc[2], axis=1, keepdims=True)
        ll_tot = jnp.sum(acc[1])

        k = jnp.minimum((_NEGPOS_RATIO * npos).astype(jnp.int32),
                        jnp.int32(_NP - 1))
        t = jnp.zeros((num, 1), jnp.int32)
        for b in range(30, 12, -1):
            cand = t | jnp.int32(1 << b)
            cnt_a = jnp.zeros((num, _C), jnp.int32)
            for gch in range(_NC):
                vb = jax.lax.bitcast_convert_type(vA[gch], jnp.int32)
                cnt_a = cnt_a + jnp.where(vb >= cand, 1, 0)
            cnt = jnp.sum(cnt_a, axis=1, keepdims=True)
            t = jnp.where(cnt >= k, cand, t)

        cntg_a = jnp.zeros((num, _C), jnp.int32)
        sumg_a = jnp.zeros((num, _C), jnp.float32)
        for gch in range(_NC):
            v = vA[gch]
            vb = jax.lax.bitcast_convert_type(v, jnp.int32)
            gt = vb > t
            cntg_a = cntg_a + jnp.where(gt, 1, 0)
            sumg_a = sumg_a + jnp.where(gt, v, 0.0)
        cnt_gt = jnp.sum(cntg_a, axis=1, keepdims=True)
        sum_gt = jnp.sum(sumg_a, axis=1, keepdims=True)
        tval = jax.lax.bitcast_convert_type(t, jnp.float32)
        topk = sum_gt + (k - cnt_gt).astype(jnp.float32) * tval
        topk = jnp.where(k > 0, topk, 0.0)
        lc_tot = jnp.sum(ce_pos + topk)
        n = jnp.sum(npos)

        lane2 = jax.lax.broadcasted_iota(jnp.int32, (1, _C), 1)
        out_ref[...] = (jnp.where(lane2 == 0, ll_tot / n, 0.0)
                        + jnp.where(lane2 == 1, lc_tot / n, 0.0))


@jax.jit
def kernel(loc_data, conf_data, priors, targets):
    num, num_priors, num_classes = conf_data.shape
    num_objs = targets.shape[1]

    conf_p = jnp.transpose(
        jnp.pad(conf_data, ((0, 0), (0, _PAD), (0, 0))).reshape(
            num, _NC, _C, num_classes), (1, 3, 0, 2))
    loc_p = jnp.transpose(
        jnp.pad(loc_data, ((0, 0), (0, _PAD), (0, 0))).reshape(
            num, _NC, _C, 4), (1, 3, 0, 2))
    priors_p = jnp.transpose(
        jnp.pad(priors, ((0, _PAD), (0, 0))).reshape(_NC, _C, 4),
        (2, 0, 1))
    truths = jnp.transpose(targets[:, :, :4], (0, 2, 1))
    labels = targets[:, :, 4]

    mbody = functools.partial(_match_body, num=num, num_objs=num_objs)
    pos_f, btid = pl.pallas_call(
        mbody,
        grid=(_NB,),
        in_specs=[
            pl.BlockSpec((num, 4, num_objs), lambda i: (0, 0, 0)),
            pl.BlockSpec((num, num_objs), lambda i: (0, 0)),
            pl.BlockSpec((4, _NC, _C), lambda i: (0, 0, 0)),
        ],
        out_specs=[
            pl.BlockSpec((_NC, num, _C), lambda i: (0, 0, 0)),
            pl.BlockSpec((_NC, num, _C), lambda i: (0, 0, 0)),
        ],
        out_shape=[
            jax.ShapeDtypeStruct((_NC, num, _C), jnp.float32),
            jax.ShapeDtypeStruct((_NC, num, _C), jnp.int32),
        ],
        scratch_shapes=[
            pltpu.VMEM((_NC, num, _C), jnp.float32),
            pltpu.VMEM((_NC, num, _C), jnp.int32),
            pltpu.VMEM((num_objs, num, _C), jnp.float32),
            pltpu.VMEM((num_objs, num, _C), jnp.int32),
        ],
    )(truths, labels, priors_p)

    lbody = functools.partial(_loss_body, num=num, num_objs=num_objs,
                              num_classes=num_classes)
    out = pl.pallas_call(
        lbody,
        grid=(_NB + 1,),
        in_specs=[
            pl.BlockSpec((num, 4, num_objs), lambda i: (0, 0, 0)),
            pl.BlockSpec((4, _NC, _C), lambda i: (0, 0, 0)),
            pl.BlockSpec((_CB, 4, num, _C),
                         lambda i: (jnp.clip(i, 0, _NB - 1), 0, 0, 0)),
            pl.BlockSpec((_CB, num_classes, num, _C),
                         lambda i: (jnp.clip(i, 0, _NB - 1), 0, 0, 0)),
            pl.BlockSpec((_NC, num, _C), lambda i: (0, 0, 0)),
            pl.BlockSpec((_NC, num, _C), lambda i: (0, 0, 0)),
        ],
        out_specs=pl.BlockSpec((1, _C), lambda i: (0, 0)),
        out_shape=jax.ShapeDtypeStruct((1, _C), jnp.float32),
        scratch_shapes=[
            pltpu.VMEM((_NC, num, _C), jnp.float32),
            pltpu.VMEM((3, num, _C), jnp.float32),
        ],
    )(truths, priors_p, loc_p, conf_p, pos_f, btid)
    return (out[0, 0], out[0, 1])


# speedup vs baseline: 1.1587x; 1.1587x over previous
import functools

import jax
import jax.numpy as jnp
from jax.experimental import pallas as pl
from jax.experimental.pallas import tpu as pltpu

_VARIANCES = (0.1, 0.2)
_THRESHOLD = 0.5
_NEGPOS_RATIO = 3

_NP = 8732
_R, _C = 72, 128
_PAD = _R * _C - _NP
_NB = 9
_BR = _R // _NB


def _body(truths_ref, labels_ref, priors_ref, loc_ref, conf_ref, out_ref,
          vA, vB, btov, btidx, pmax, pminidx, bp, part,
          num, num_objs, num_classes):
    i = pl.program_id(0)

    @pl.when(i < _NB)
    def _phase1():
        sl = i
        x = conf_ref[...]
        s = jnp.sum(jnp.exp(x), axis=1)
        lse = jnp.log(s)
        vA[:, pl.ds(sl * _BR, _BR), :] = lse - x[:, 0]
        vB[:, pl.ds(sl * _BR, _BR), :] = lse - x[:, 1]

        pcx = priors_ref[0]
        pcy = priors_ref[1]
        pw = priors_ref[2]
        ph = priors_ref[3]
        px1 = pcx - pw / 2.0
        py1 = pcy - ph / 2.0
        px2 = pcx + pw / 2.0
        py2 = pcy + ph / 2.0
        parea = (px2 - px1) * (py2 - py1)

        idx_blk = (sl * _BR * _C
                   + jax.lax.broadcasted_iota(jnp.int32, (_BR, _C), 0) * _C
                   + jax.lax.broadcasted_iota(jnp.int32, (_BR, _C), 1))

        bt_ov = jnp.full((num, _BR, _C), -1.0, dtype=jnp.float32)
        bt_id = jnp.zeros((num, _BR, _C), dtype=jnp.int32)
        for j in range(num_objs):
            tx1 = truths_ref[:, 0, j][:, None, None]
            ty1 = truths_ref[:, 1, j][:, None, None]
            tx2 = truths_ref[:, 2, j][:, None, None]
            ty2 = truths_ref[:, 3, j][:, None, None]
            ix = jnp.clip(jnp.minimum(px2, tx2) - jnp.maximum(px1, tx1),
                          0.0, None)
            iy = jnp.clip(jnp.minimum(py2, ty2) - jnp.maximum(py1, ty1),
                          0.0, None)
            inter = ix * iy
            tarea = (tx2 - tx1) * (ty2 - ty1)
            ov = inter / (tarea + parea - inter)
            upd = ov > bt_ov
            bt_ov = jnp.where(upd, ov, bt_ov)
            bt_id = jnp.where(upd, j, bt_id)
            mj = jnp.max(ov, axis=(1, 2))
            mn = jnp.min(jnp.where(ov == mj[:, None, None], idx_blk,
                                   jnp.int32(2**30)), axis=(1, 2))
            pmax[sl, j, :] = mj
            pminidx[sl, j, :] = mn
        btov[:, pl.ds(sl * _BR, _BR), :] = bt_ov
        btidx[:, pl.ds(sl * _BR, _BR), :] = bt_id

        @pl.when(sl == _NB - 1)
        def _finalize_bp():
            pm = pmax[...]
            pi = pminidx[...]
            gmax = jnp.max(pm, axis=0)
            bp[...] = jnp.min(jnp.where(pm == gmax[None], pi,
                                        jnp.int32(2**30)), axis=0)

    @pl.when((i >= _NB) & (i < 2 * _NB))
    def _phase2():
        sl = i - _NB

        pcx = priors_ref[0]
        pcy = priors_ref[1]
        pw = priors_ref[2]
        ph = priors_ref[3]

        idx_blk = (sl * _BR * _C
                   + jax.lax.broadcasted_iota(jnp.int32, (_BR, _C), 0) * _C
                   + jax.lax.broadcasted_iota(jnp.int32, (_BR, _C), 1))

        bt_ov = btov[:, pl.ds(sl * _BR, _BR), :]
        bt_id = btidx[:, pl.ds(sl * _BR, _BR), :]
        for j in range(num_objs):
            eq = idx_blk[None] == bp[j, :][:, None, None]
            bt_ov = jnp.where(eq, 2.0, bt_ov)
            bt_id = jnp.where(eq, j, bt_id)

        conf_t = jnp.zeros((num, _BR, _C), dtype=jnp.float32)
        for j in range(num_objs):
            conf_t = jnp.where(bt_id == j, labels_ref[:, j][:, None, None],
                               conf_t)
        conf_t = jnp.where(bt_ov < _THRESHOLD, 0.0, conf_t)
        pos = (conf_t > 0.0) & (idx_blk[None] < _NP)
        part[0, sl, :] = jnp.sum(pos.astype(jnp.float32), axis=(1, 2))

        mt = []
        for c in range(4):
            acc = jnp.zeros((num, _BR, _C), dtype=jnp.float32)
            for j in range(num_objs):
                acc = jnp.where(bt_id == j, truths_ref[:, c, j][:, None, None],
                                acc)
            mt.append(acc)
        mx1, my1, mx2, my2 = mt
        g = (((mx1 + mx2) / 2.0 - pcx) / (_VARIANCES[0] * pw),
             ((my1 + my2) / 2.0 - pcy) / (_VARIANCES[0] * ph),
             jnp.log((mx2 - mx1) / pw) / _VARIANCES[1],
             jnp.log((my2 - my1) / ph) / _VARIANCES[1])
        ll = jnp.zeros((num,), dtype=jnp.float32)
        for c in range(4):
            d = loc_ref[:, c] - g[c]
            ad = jnp.abs(d)
            sl1 = jnp.where(ad < 1.0, 0.5 * d * d, ad - 0.5)
            ll += jnp.sum(jnp.where(pos, sl1, 0.0), axis=(1, 2))
        part[1, sl, :] = ll

        vb_blk = vB[:, pl.ds(sl * _BR, _BR), :]
        part[2, sl, :] = jnp.sum(jnp.where(pos, vb_blk, 0.0), axis=(1, 2))
        va_blk = vA[:, pl.ds(sl * _BR, _BR), :]
        vA[:, pl.ds(sl * _BR, _BR), :] = jnp.maximum(
            jnp.where(pos | (idx_blk[None] >= _NP), 0.0, va_blk), 0.0)

    @pl.when(i == 2 * _NB)
    def _phase3():
        npos = jnp.sum(part[0], axis=0)
        ll_tot = jnp.sum(part[1])
        ce_pos = jnp.sum(part[2], axis=0)

        k = jnp.minimum((_NEGPOS_RATIO * npos).astype(jnp.int32),
                        jnp.int32(_NP - 1))
        v = vA[...]
        vb = jax.lax.bitcast_convert_type(v, jnp.int32)
        t = jnp.zeros((num,), dtype=jnp.int32)
        for b in range(30, 12, -1):
            cand = t | jnp.int32(1 << b)
            cnt = jnp.sum((vb >= cand[:, None, None]).astype(jnp.int32),
                          axis=(1, 2))
            t = jnp.where(cnt >= k, cand, t)
        cnt_gt = jnp.sum((vb > t[:, None, None]).astype(jnp.int32),
                         axis=(1, 2))
        sum_gt = jnp.sum(jnp.where(vb > t[:, None, None], v, 0.0),
                         axis=(1, 2))
        tval = jax.lax.bitcast_convert_type(t, jnp.float32)
        topk = sum_gt + (k - cnt_gt).astype(jnp.float32) * tval
        topk = jnp.where(k > 0, topk, 0.0)
        lc_tot = jnp.sum(ce_pos + topk)
        n = jnp.sum(npos)

        lane = jax.lax.broadcasted_iota(jnp.int32, (1, 128), 1)
        out_ref[...] = (jnp.where(lane == 0, ll_tot / n, 0.0)
                        + jnp.where(lane == 1, lc_tot / n, 0.0))


@jax.jit
def kernel(loc_data, conf_data, priors, targets):
    num, num_priors, num_classes = conf_data.shape
    num_objs = targets.shape[1]

    conf_p = jnp.pad(jnp.transpose(conf_data, (0, 2, 1)),
                     ((0, 0), (0, 0), (0, _PAD))).reshape(
                         num, num_classes, _R, _C)
    loc_p = jnp.pad(jnp.transpose(loc_data, (0, 2, 1)),
                    ((0, 0), (0, 0), (0, _PAD))).reshape(num, 4, _R, _C)
    priors_p = jnp.pad(priors.T, ((0, 0), (0, _PAD))).reshape(4, _R, _C)
    truths = jnp.transpose(targets[:, :, :4], (0, 2, 1))
    labels = targets[:, :, 4]

    body = functools.partial(_body, num=num, num_objs=num_objs,
                             num_classes=num_classes)
    out = pl.pallas_call(
        body,
        grid=(2 * _NB + 1,),
        in_specs=[
            pl.BlockSpec((num, 4, num_objs), lambda i: (0, 0, 0)),
            pl.BlockSpec((num, num_objs), lambda i: (0, 0)),
            pl.BlockSpec((4, _BR, _C),
                         lambda i: (0, jnp.where(i < _NB, i,
                                                 jnp.clip(i - _NB, 0, _NB - 1)),
                                    0)),
            pl.BlockSpec((num, 4, _BR, _C),
                         lambda i: (0, 0, jnp.clip(i - _NB, 0, _NB - 1), 0)),
            pl.BlockSpec((num, num_classes, _BR, _C),
                         lambda i: (0, 0, jnp.clip(i, 0, _NB - 1), 0)),
        ],
        out_specs=pl.BlockSpec((1, 128), lambda i: (0, 0)),
        out_shape=jax.ShapeDtypeStruct((1, 128), jnp.float32),
        scratch_shapes=[
            pltpu.VMEM((num, _R, _C), jnp.float32),
            pltpu.VMEM((num, _R, _C), jnp.float32),
            pltpu.VMEM((num, _R, _C), jnp.float32),
            pltpu.VMEM((num, _R, _C), jnp.int32),
            pltpu.VMEM((_NB, num_objs, num), jnp.float32),
            pltpu.VMEM((_NB, num_objs, num), jnp.int32),
            pltpu.VMEM((num_objs, num), jnp.int32),
            pltpu.VMEM((3, _NB, num), jnp.float32),
        ],
    )(truths, labels, priors_p, loc_p, conf_p)
    return (out[0, 0], out[0, 1])
